# baseline (speedup 1.0000x reference)
import functools

import jax
import jax.numpy as jnp
from jax import lax
from jax.experimental import pallas as pl
from jax.experimental.pallas import tpu as pltpu
from jax.experimental.pallas import tpu_sc as plsc

_GATHER_BLOCK = 8
_CHUNK = 4


def _sc_assemble(x, emb, out_sds):
    b, t, d = x.shape
    e = emb.shape[1]
    try:
        info = plsc.get_sparse_core_info()
        num_cores, num_subcores = info.num_cores, info.num_subcores
    except Exception:
        num_cores, num_subcores = 2, 16
    num_workers = num_cores * num_subcores
    b_per_w = b // num_workers
    ch = _CHUNK
    mesh = plsc.VectorSubcoreMesh(core_axis_name="c", subcore_axis_name="s")

    @functools.partial(
        pl.kernel,
        out_type=out_sds,
        mesh=mesh,
        scratch_types=[
            pltpu.VMEM((e,), jnp.float32),
            *[pltpu.VMEM((t, e), jnp.float32) for _ in range(ch)],
            pltpu.SemaphoreType.DMA,
            pltpu.SemaphoreType.DMA,
        ],
    )
    def assemble(x_hbm, emb_hbm, out_hbm, stage_v, *rest):
        bcs, (sem_x, sem_b) = rest[:ch], rest[ch:]
        wid = lax.axis_index("s") * num_cores + lax.axis_index("c")
        base = wid * b_per_w

        def chunk_body(ci, carry):
            row0 = base + ci * ch
            x_cps = [
                pltpu.async_copy(
                    x_hbm.at[row0 + j],
                    out_hbm.at[row0 + j, :, pl.ds(0, d)],
                    sem_x,
                )
                for j in range(ch)
            ]
            b_cps = []
            for j in range(ch):
                row = row0 + j
                pltpu.sync_copy(emb_hbm.at[row], stage_v)
                vs = [stage_v[pl.ds(16 * c, 16)] for c in range(e // 16)]
                bc = bcs[j]
                for tt in range(t):
                    for c in range(e // 16):
                        bc[tt, pl.ds(16 * c, 16)] = vs[c]
                b_cps.append(
                    pltpu.async_copy(
                        bc, out_hbm.at[row, :, pl.ds(d, e)], sem_b
                    )
                )
            for cp in b_cps:
                cp.wait()
            for cp in x_cps:
                cp.wait()
            return carry

        lax.fori_loop(0, b_per_w // ch, chunk_body, 0)

    return assemble(x, emb)


def _gather_body(sid_ref, *refs):
    g = _GATHER_BLOCK
    table_refs, emb_ref = refs[:-1], refs[-1]
    i = pl.program_id(0)
    for k, tref in enumerate(table_refs):
        r = sid_ref[i * g + k] % 8
        emb_ref[k, :] = tref[pl.ds(r, 1), :][0, :]


def kernel(x, speaker_id, emb_table):
    b, t, d = x.shape
    e = emb_table.shape[1]
    g = _GATHER_BLOCK
    out_sds = jax.ShapeDtypeStruct((b, t, d + e), jnp.float32)

    emb = pl.pallas_call(
        _gather_body,
        grid_spec=pltpu.PrefetchScalarGridSpec(
            num_scalar_prefetch=1,
            grid=(b // g,),
            in_specs=[
                pl.BlockSpec((8, e), functools.partial(
                    lambda k, i, sid: (sid[i * g + k] // 8, 0), k))
                for k in range(g)
            ],
            out_specs=pl.BlockSpec((g, e), lambda i, sid: (i, 0)),
        ),
        out_shape=jax.ShapeDtypeStruct((b, e), jnp.float32),
    )(speaker_id.astype(jnp.int32), *([emb_table] * g))

    return _sc_assemble(x, emb, out_sds)


# speedup vs baseline: 8.2832x; 8.2832x over previous
import functools

import jax
import jax.numpy as jnp
from jax import lax
from jax.experimental import pallas as pl
from jax.experimental.pallas import tpu as pltpu
from jax.experimental.pallas import tpu_sc as plsc

_GATHER_BLOCK = 8
_CHUNK = 2


def _sc_assemble(x, emb, out_sds):
    b, t, d = x.shape
    e = emb.shape[1]
    try:
        info = plsc.get_sparse_core_info()
        num_cores, num_subcores = info.num_cores, info.num_subcores
    except Exception:
        num_cores, num_subcores = 2, 16
    num_workers = num_cores * num_subcores
    b_per_w = b // num_workers
    ch = _CHUNK
    mesh = plsc.VectorSubcoreMesh(core_axis_name="c", subcore_axis_name="s")

    @functools.partial(
        pl.kernel,
        out_type=out_sds,
        mesh=mesh,
        scratch_types=[
            pltpu.VMEM((b_per_w, e), jnp.float32),
            *[pltpu.VMEM((t, d), jnp.float32) for _ in range(ch)],
            *[pltpu.VMEM((t, e), jnp.float32) for _ in range(ch)],
            pltpu.SemaphoreType.DMA,
            pltpu.SemaphoreType.DMA,
        ],
    )
    def assemble(x_hbm, emb_hbm, out_hbm, emb_v, *rest):
        xbs, bcs = rest[:ch], rest[ch:2 * ch]
        sem_in, sem_out = rest[2 * ch:]
        wid = lax.axis_index("s") * num_cores + lax.axis_index("c")
        base = wid * b_per_w
        pltpu.sync_copy(emb_hbm.at[pl.ds(base, b_per_w)], emb_v)

        def chunk_body(ci, carry):
            row0 = base + ci * ch
            cps_in = [
                pltpu.async_copy(x_hbm.at[row0 + j], xbs[j], sem_in)
                for j in range(ch)
            ]
            cps_out = []
            for j in range(ch):
                r = ci * ch + j
                vs = [emb_v[r, pl.ds(16 * c, 16)] for c in range(e // 16)]
                bc = bcs[j]
                for tt in range(t):
                    for c in range(e // 16):
                        bc[tt, pl.ds(16 * c, 16)] = vs[c]
                cps_in[j].wait()
                cps_out.append(pltpu.async_copy(
                    xbs[j], out_hbm.at[row0 + j, :, pl.ds(0, d)], sem_out))
                cps_out.append(pltpu.async_copy(
                    bc, out_hbm.at[row0 + j, :, pl.ds(d, e)], sem_out))
            for cp in cps_out:
                cp.wait()
            return carry

        lax.fori_loop(0, b_per_w // ch, chunk_body, 0)

    return assemble(x, emb)


def _gather_body(sid_ref, *refs):
    g = _GATHER_BLOCK
    table_refs, emb_ref = refs[:-1], refs[-1]
    i = pl.program_id(0)
    for k, tref in enumerate(table_refs):
        r = sid_ref[i * g + k] % 8
        emb_ref[k, :] = tref[pl.ds(r, 1), :][0, :]


def kernel(x, speaker_id, emb_table):
    b, t, d = x.shape
    e = emb_table.shape[1]
    g = _GATHER_BLOCK
    out_sds = jax.ShapeDtypeStruct((b, t, d + e), jnp.float32)

    emb = pl.pallas_call(
        _gather_body,
        grid_spec=pltpu.PrefetchScalarGridSpec(
            num_scalar_prefetch=1,
            grid=(b // g,),
            in_specs=[
                pl.BlockSpec((8, e), functools.partial(
                    lambda k, i, sid: (sid[i * g + k] // 8, 0), k))
                for k in range(g)
            ],
            out_specs=pl.BlockSpec((g, e), lambda i, sid: (i, 0)),
        ),
        out_shape=jax.ShapeDtypeStruct((b, e), jnp.float32),
    )(speaker_id.astype(jnp.int32), *([emb_table] * g))

    return _sc_assemble(x, emb, out_sds)


# speedup vs baseline: 8.3017x; 1.0022x over previous
import functools

import jax
import jax.numpy as jnp
from jax import lax
from jax.experimental import pallas as pl
from jax.experimental.pallas import tpu as pltpu
from jax.experimental.pallas import tpu_sc as plsc

_GATHER_BLOCK = 8
_CHUNK = 2


def _sc_assemble(x, emb, out_sds):
    b, t, d = x.shape
    e = emb.shape[1]
    try:
        info = plsc.get_sparse_core_info()
        num_cores, num_subcores = info.num_cores, info.num_subcores
    except Exception:
        num_cores, num_subcores = 2, 16
    num_workers = num_cores * num_subcores
    b_per_w = b // num_workers
    ch = _CHUNK
    mesh = plsc.VectorSubcoreMesh(core_axis_name="c", subcore_axis_name="s")

    @functools.partial(
        pl.kernel,
        out_type=out_sds,
        mesh=mesh,
        scratch_types=[
            pltpu.VMEM((b_per_w, e), jnp.float32),
            *[pltpu.VMEM((t, d + e), jnp.float32) for _ in range(ch)],
            pltpu.SemaphoreType.DMA,
            pltpu.SemaphoreType.DMA,
        ],
    )
    def assemble(x_hbm, emb_hbm, out_hbm, emb_v, *rest):
        combs = rest[:ch]
        sem_in, sem_out = rest[ch:]
        wid = lax.axis_index("s") * num_cores + lax.axis_index("c")
        base = wid * b_per_w
        pltpu.sync_copy(emb_hbm.at[pl.ds(base, b_per_w)], emb_v)

        def slot(u, j):
            row = base + u
            comb = combs[j]

            @pl.when(u >= ch)
            def _():
                pltpu.make_async_copy(comb, out_hbm.at[row - ch], sem_out).wait()

            cp_in = pltpu.async_copy(
                x_hbm.at[row], comb.at[:, pl.ds(0, d)], sem_in)
            vs = [emb_v[u, pl.ds(16 * c, 16)] for c in range(e // 16)]
            for tt in range(t):
                for c in range(e // 16):
                    comb[tt, pl.ds(d + 16 * c, 16)] = vs[c]
            cp_in.wait()
            pltpu.async_copy(comb, out_hbm.at[row], sem_out)

        def g_body(g, carry):
            for j in range(ch):
                slot(g * ch + j, j)
            return carry

        lax.fori_loop(0, b_per_w // ch, g_body, 0)
        for j in range(ch):
            u = b_per_w - ch + j
            pltpu.make_async_copy(
                combs[j], out_hbm.at[base + u], sem_out).wait()

    return assemble(x, emb)


def _gather_body(sid_ref, *refs):
    g = _GATHER_BLOCK
    table_refs, emb_ref = refs[:-1], refs[-1]
    i = pl.program_id(0)
    for k, tref in enumerate(table_refs):
        r = sid_ref[i * g + k] % 8
        emb_ref[k, :] = tref[pl.ds(r, 1), :][0, :]


def kernel(x, speaker_id, emb_table):
    b, t, d = x.shape
    e = emb_table.shape[1]
    g = _GATHER_BLOCK
    out_sds = jax.ShapeDtypeStruct((b, t, d + e), jnp.float32)

    emb = pl.pallas_call(
        _gather_body,
        grid_spec=pltpu.PrefetchScalarGridSpec(
            num_scalar_prefetch=1,
            grid=(b // g,),
            in_specs=[
                pl.BlockSpec((8, e), functools.partial(
                    lambda k, i, sid: (sid[i * g + k] // 8, 0), k))
                for k in range(g)
            ],
            out_specs=pl.BlockSpec((g, e), lambda i, sid: (i, 0)),
        ),
        out_shape=jax.ShapeDtypeStruct((b, e), jnp.float32),
    )(speaker_id.astype(jnp.int32), *([emb_table] * g))

    return _sc_assemble(x, emb, out_sds)
